# baseline (speedup 1.0000x reference)
import jax
import jax.numpy as jnp
from jax.experimental import pallas as pl
from jax.experimental.pallas import tpu as pltpu

_F32 = jnp.float32
_BF16 = jnp.bfloat16


def _conv1x1_manual_kernel(w_ref, x_hbm, o_ref, xbuf, sem):
    i = pl.program_id(0)
    n = pl.num_programs(0)
    slot = jax.lax.rem(i, 2)
    nxt = jax.lax.rem(i + 1, 2)

    @pl.when(i == 0)
    def _():
        pltpu.make_async_copy(x_hbm.at[0], xbuf.at[0], sem.at[0]).start()

    @pl.when(i + 1 < n)
    def _():
        pltpu.make_async_copy(
            x_hbm.at[i + 1], xbuf.at[nxt], sem.at[nxt]
        ).start()

    pltpu.make_async_copy(x_hbm.at[i], xbuf.at[slot], sem.at[slot]).wait()

    wb = w_ref[...].astype(_BF16)
    xb = xbuf[slot].astype(_BF16)
    o_ref[0] = jnp.dot(wb, xb, preferred_element_type=_F32)


def kernel(x_nchw, w2d):
    N, C, H, W = x_nchw.shape
    COUT, CIN = w2d.shape
    HW = H * W
    x3d = x_nchw.reshape(N, CIN, HW)

    out3d = pl.pallas_call(
        _conv1x1_manual_kernel,
        out_shape=jax.ShapeDtypeStruct((N, COUT, HW), _F32),
        grid=(N,),
        in_specs=[
            pl.BlockSpec((COUT, CIN), lambda i: (0, 0)),
            pl.BlockSpec(memory_space=pl.ANY),
        ],
        out_specs=pl.BlockSpec((1, COUT, HW), lambda i: (i, 0, 0)),
        scratch_shapes=[
            pltpu.VMEM((2, CIN, HW), _F32),
            pltpu.SemaphoreType.DMA((2,)),
        ],
        compiler_params=pltpu.CompilerParams(
            dimension_semantics=("arbitrary",),
            vmem_limit_bytes=60 * 2**20,
        ),
        cost_estimate=pl.CostEstimate(
            flops=2 * N * HW * CIN * COUT,
            transcendentals=0,
            bytes_accessed=(N * CIN * HW + COUT * CIN + N * COUT * HW) * 4,
        ),
    )(w2d, x3d)
    return out3d.reshape(N, COUT, H, W)


# speedup vs baseline: 1.0056x; 1.0056x over previous
import jax
import jax.numpy as jnp
from jax.experimental import pallas as pl
from jax.experimental.pallas import tpu as pltpu

_F32 = jnp.float32
_BF16 = jnp.bfloat16


def _conv1x1_kernel(w_ref, x_ref, o_ref):
    wb = w_ref[...].astype(_BF16)
    xb = x_ref[0].astype(_BF16)
    o_ref[0] = jnp.dot(wb, xb, preferred_element_type=_F32)


def _pick_tile(hw, cap=4096):
    if hw % 128 != 0:
        return hw
    for t in range(min(cap, hw), 127, -128):
        if hw % t == 0:
            return t
    return hw


def kernel(x_nchw, w2d):
    N, C, H, W = x_nchw.shape
    COUT, CIN = w2d.shape
    HW = H * W
    x3d = x_nchw.reshape(N, CIN, HW)

    tile = _pick_tile(HW)
    s = HW // tile

    x_bytes = CIN * tile * 4
    o_bytes = COUT * tile * 4
    vmem = int(min(112 * 2**20,
                   2 * (x_bytes + o_bytes) + COUT * CIN * 4 + (8 << 20)))

    out3d = pl.pallas_call(
        _conv1x1_kernel,
        out_shape=jax.ShapeDtypeStruct((N, COUT, HW), _F32),
        grid=(N, s),
        in_specs=[
            pl.BlockSpec((COUT, CIN), lambda n, t: (0, 0)),
            pl.BlockSpec((1, CIN, tile), lambda n, t: (n, 0, t)),
        ],
        out_specs=pl.BlockSpec((1, COUT, tile), lambda n, t: (n, 0, t)),
        compiler_params=pltpu.CompilerParams(
            dimension_semantics=("parallel", "parallel"),
            vmem_limit_bytes=vmem,
        ),
        cost_estimate=pl.CostEstimate(
            flops=2 * N * HW * CIN * COUT,
            transcendentals=0,
            bytes_accessed=(N * CIN * HW + COUT * CIN + N * COUT * HW) * 4,
        ),
    )(w2d, x3d)
    return out3d.reshape(N, COUT, H, W)
